# baseline (speedup 1.0000x reference)
import functools

import jax
import jax.numpy as jnp
from jax.experimental import pallas as pl
from jax.experimental.pallas import tpu as pltpu


def _round_up(x, m):
    return ((x + m - 1) // m) * m


def _pick_tile(n, pref):
    for t in (pref, 512, 256, 128, 64, 32, 16, 8):
        if t <= pref and n % t == 0:
            return t
    return n


def _mega_kernel(adj_lo_ref, adj_hi_ref, x_ref, w1_ref, b1_ref, w2_ref,
                 b2_ref, o_ref, adjb_s, d_s, u_s, t2_s, *, tm, nb):
    p = pl.program_id(0)
    k = pl.program_id(1)
    start = pl.multiple_of(k * tm, tm)

    @pl.when(p == 0)
    def _phase0():
        a_lo = adj_lo_ref[...]
        a_hi = adj_hi_ref[...]
        half = a_lo.shape[1]
        row = jax.lax.broadcasted_iota(jnp.int32, (tm, half), 0)
        col = jax.lax.broadcasted_iota(jnp.int32, (tm, half), 1)
        ab_lo = jnp.where(col == row + start, jnp.bfloat16(1.0),
                          a_lo.astype(jnp.bfloat16))
        ab_hi = jnp.where(col == row + (start - half), jnp.bfloat16(1.0),
                          a_hi.astype(jnp.bfloat16))
        adjb_s[pl.ds(start, tm), :half] = ab_lo
        adjb_s[pl.ds(start, tm), half:] = ab_hi
        deg = (jnp.sum(a_lo, axis=1, keepdims=True)
               + jnp.sum(a_hi, axis=1, keepdims=True) + 1.0)
        dk = jax.lax.rsqrt(deg)
        d_s[pl.ds(start, tm), :] = dk
        s1 = jnp.dot(x_ref[...].astype(jnp.bfloat16), w1_ref[...],
                     preferred_element_type=jnp.float32)
        t1k = (s1 * dk).astype(jnp.bfloat16)
        c_lo = jax.lax.dot_general(
            ab_lo, t1k, (((0,), (0,)), ((), ())),
            preferred_element_type=jnp.float32)
        c_hi = jax.lax.dot_general(
            ab_hi, t1k, (((0,), (0,)), ((), ())),
            preferred_element_type=jnp.float32)

        @pl.when(k == 0)
        def _init():
            u_s[:half, :] = c_lo
            u_s[half:, :] = c_hi

        @pl.when(k > 0)
        def _acc():
            u_s[:half, :] += c_lo
            u_s[half:, :] += c_hi

    @pl.when(p == 1)
    def _phase1():
        @pl.when(k == 0)
        def _compute_t2():
            d_all = d_s[...]
            h = jnp.maximum(d_all * u_s[...] + b1_ref[...], 0.0)
            s2 = jnp.dot(h.astype(jnp.bfloat16), w2_ref[...],
                         preferred_element_type=jnp.float32)
            t2_s[...] = (d_all * s2).astype(jnp.bfloat16)

        hm = tm // 2
        t2 = t2_s[...]
        a_lo = adjb_s[pl.ds(start, hm), :]
        acc_lo = jnp.dot(a_lo, t2, preferred_element_type=jnp.float32)
        o_ref[:hm, :] = d_s[pl.ds(start, hm), :] * acc_lo + b2_ref[...]
        a_hi = adjb_s[pl.ds(start + hm, hm), :]
        acc_hi = jnp.dot(a_hi, t2, preferred_element_type=jnp.float32)
        o_ref[hm:, :] = d_s[pl.ds(start + hm, hm), :] * acc_hi + b2_ref[...]


def kernel(adj, x, w1, b1, w2, b2):
    n = adj.shape[0]
    f_in, h_dim = w1.shape
    c_dim = w2.shape[1]
    fp = _round_up(f_in, 128)
    hp = _round_up(h_dim, 128)
    tm = _pick_tile(n, 512)
    nb = n // tm
    f32 = jnp.float32
    bf16 = jnp.bfloat16

    if f_in != fp or h_dim != hp:
        w1_in = jnp.zeros((fp, hp), f32).at[:f_in, :h_dim].set(w1)
    else:
        w1_in = w1
    x_in = x if f_in == fp else jnp.zeros((n, fp), f32).at[:, :f_in].set(x)
    w1_in = w1_in.astype(bf16)
    if h_dim != hp:
        w2 = jnp.zeros((hp, c_dim), f32).at[:h_dim, :].set(w2)
        b1 = jnp.zeros((hp,), f32).at[:h_dim].set(b1.astype(f32))
    w2_in = w2.astype(bf16)
    b1_2d = b1.reshape(1, hp).astype(f32)
    b2_2d = b2.reshape(1, c_dim).astype(f32)

    mib = 1 << 20

    out = pl.pallas_call(
        functools.partial(_mega_kernel, tm=tm, nb=nb),
        out_shape=jax.ShapeDtypeStruct((n, c_dim), f32),
        grid_spec=pltpu.PrefetchScalarGridSpec(
            num_scalar_prefetch=0,
            grid=(2, nb),
            in_specs=[
                pl.BlockSpec((tm, n // 2), lambda p, k: (jnp.where(p == 0, k, nb - 1), 0)),
                pl.BlockSpec((tm, n // 2), lambda p, k: (jnp.where(p == 0, k, nb - 1), 1)),
                pl.BlockSpec((tm, fp), lambda p, k: (jnp.where(p == 0, k, nb - 1), 0)),
                pl.BlockSpec((fp, hp), lambda p, k: (0, 0)),
                pl.BlockSpec((1, hp), lambda p, k: (0, 0)),
                pl.BlockSpec((hp, c_dim), lambda p, k: (0, 0)),
                pl.BlockSpec((1, c_dim), lambda p, k: (0, 0)),
            ],
            out_specs=pl.BlockSpec((tm, c_dim), lambda p, k: (jnp.where(p == 1, k, 0), 0)),
            scratch_shapes=[
                pltpu.VMEM((n, n), bf16),
                pltpu.VMEM((n, 1), f32),
                pltpu.VMEM((n, hp), f32),
                pltpu.VMEM((n, c_dim), bf16),
            ],
        ),
        compiler_params=pltpu.CompilerParams(
            dimension_semantics=("arbitrary", "arbitrary"),
            vmem_limit_bytes=63 * mib,
        ),
    )(adj, adj, x_in, w1_in, b1_2d, w2_in, b2_2d)

    return out


# speedup vs baseline: 1.0508x; 1.0508x over previous
import functools

import jax
import jax.numpy as jnp
from jax.experimental import pallas as pl
from jax.experimental.pallas import tpu as pltpu


def _round_up(x, m):
    return ((x + m - 1) // m) * m


def _pick_tile(n, pref):
    for t in (pref, 512, 256, 128, 64, 32, 16, 8):
        if t <= pref and n % t == 0:
            return t
    return n


def _mega_kernel(adj_ref, x_ref, w1_ref, b1_ref, w2_ref, b2_ref, o_ref,
                 adjb_s, d_s, u_s, t2_s, *, tm, nb):
    p = pl.program_id(0)
    k = pl.program_id(1)
    start = pl.multiple_of(k * tm, tm)

    @pl.when(p == 0)
    def _phase0():
        a = adj_ref[...]
        n = a.shape[1]
        row = jax.lax.broadcasted_iota(jnp.int32, (tm, n), 0)
        col = jax.lax.broadcasted_iota(jnp.int32, (tm, n), 1)
        ab = jnp.where(col == row + start, jnp.bfloat16(1.0),
                       a.astype(jnp.bfloat16))
        adjb_s[pl.ds(start, tm), :] = ab
        deg = jnp.sum(a, axis=1, keepdims=True) + 1.0
        dk = jax.lax.rsqrt(deg)
        d_s[pl.ds(start, tm), :] = dk
        s1 = jnp.dot(x_ref[...].astype(jnp.bfloat16),
                     w1_ref[...].astype(jnp.bfloat16),
                     preferred_element_type=jnp.float32)
        t1k = (s1 * dk).astype(jnp.bfloat16)
        half = n // 2
        c_lo = jax.lax.dot_general(
            ab[:, :half], t1k, (((0,), (0,)), ((), ())),
            preferred_element_type=jnp.float32)
        c_hi = jax.lax.dot_general(
            ab[:, half:], t1k, (((0,), (0,)), ((), ())),
            preferred_element_type=jnp.float32)

        @pl.when(k == 0)
        def _init():
            u_s[:half, :] = c_lo
            u_s[half:, :] = c_hi

        @pl.when(k > 0)
        def _acc():
            u_s[:half, :] += c_lo
            u_s[half:, :] += c_hi

    @pl.when(p == 1)
    def _phase1():
        @pl.when(k == 0)
        def _compute_t2():
            d_all = d_s[...]
            h = jnp.maximum(d_all * u_s[...] + b1_ref[...], 0.0)
            s2 = jnp.dot(h.astype(jnp.bfloat16),
                         w2_ref[...].astype(jnp.bfloat16),
                         preferred_element_type=jnp.float32)
            t2_s[...] = (d_all * s2).astype(jnp.bfloat16)

        hm = tm // 2
        t2 = t2_s[...]
        a_lo = adjb_s[pl.ds(start, hm), :]
        acc_lo = jnp.dot(a_lo, t2, preferred_element_type=jnp.float32)
        o_ref[:hm, :] = d_s[pl.ds(start, hm), :] * acc_lo + b2_ref[...]
        a_hi = adjb_s[pl.ds(start + hm, hm), :]
        acc_hi = jnp.dot(a_hi, t2, preferred_element_type=jnp.float32)
        o_ref[hm:, :] = d_s[pl.ds(start + hm, hm), :] * acc_hi + b2_ref[...]


def kernel(adj, x, w1, b1, w2, b2):
    n = adj.shape[0]
    f_in, h_dim = w1.shape
    c_dim = w2.shape[1]
    fp = _round_up(f_in, 128)
    hp = _round_up(h_dim, 128)
    tm = _pick_tile(n, 512)
    nb = n // tm
    f32 = jnp.float32
    bf16 = jnp.bfloat16

    if f_in != fp or h_dim != hp:
        w1_in = jnp.zeros((fp, hp), f32).at[:f_in, :h_dim].set(w1)
    else:
        w1_in = w1
    x_in = x if f_in == fp else jnp.zeros((n, fp), f32).at[:, :f_in].set(x)
    if h_dim != hp:
        w2 = jnp.zeros((hp, c_dim), f32).at[:h_dim, :].set(w2)
        b1 = jnp.zeros((hp,), f32).at[:h_dim].set(b1.astype(f32))
    w2_in = w2
    b1_2d = b1.reshape(1, hp).astype(f32)
    b2_2d = b2.reshape(1, c_dim).astype(f32)

    mib = 1 << 20

    out = pl.pallas_call(
        functools.partial(_mega_kernel, tm=tm, nb=nb),
        out_shape=jax.ShapeDtypeStruct((n, c_dim), f32),
        grid_spec=pltpu.PrefetchScalarGridSpec(
            num_scalar_prefetch=0,
            grid=(2, nb),
            in_specs=[
                pl.BlockSpec((tm, n), lambda p, k: (jnp.where(p == 0, k, nb - 1), 0)),
                pl.BlockSpec((tm, fp), lambda p, k: (jnp.where(p == 0, k, nb - 1), 0)),
                pl.BlockSpec((fp, hp), lambda p, k: (0, 0)),
                pl.BlockSpec((1, hp), lambda p, k: (0, 0)),
                pl.BlockSpec((hp, c_dim), lambda p, k: (0, 0)),
                pl.BlockSpec((1, c_dim), lambda p, k: (0, 0)),
            ],
            out_specs=pl.BlockSpec((tm, c_dim), lambda p, k: (jnp.where(p == 1, k, 0), 0)),
            scratch_shapes=[
                pltpu.VMEM((n, n), bf16),
                pltpu.VMEM((n, 1), f32),
                pltpu.VMEM((n, hp), f32),
                pltpu.VMEM((n, c_dim), bf16),
            ],
        ),
        compiler_params=pltpu.CompilerParams(
            dimension_semantics=("arbitrary", "arbitrary"),
            vmem_limit_bytes=63 * mib,
        ),
    )(adj, x_in, w1_in, b1_2d, w2_in, b2_2d)

    return out


# speedup vs baseline: 1.0548x; 1.0038x over previous
import functools

import jax
import jax.numpy as jnp
from jax.experimental import pallas as pl
from jax.experimental.pallas import tpu as pltpu


def _round_up(x, m):
    return ((x + m - 1) // m) * m


def _pick_tile(n, pref):
    for t in (pref, 512, 256, 128, 64, 32, 16, 8):
        if t <= pref and n % t == 0:
            return t
    return n


def _mega_kernel(adj_ref, x_ref, w1_ref, b1_ref, w2_ref, b2_ref, o_ref,
                 adjb_s, d_s, u_s, t2_s, *, tm, nb):
    p = pl.program_id(0)
    k = pl.program_id(1)
    start = pl.multiple_of(k * tm, tm)

    @pl.when(p == 0)
    def _phase0():
        a = adj_ref[...]
        n = a.shape[1]
        row = jax.lax.broadcasted_iota(jnp.int32, (tm, n), 0)
        col = jax.lax.broadcasted_iota(jnp.int32, (tm, n), 1)
        ab = jnp.where(col == row + start, jnp.bfloat16(1.0),
                       a.astype(jnp.bfloat16))
        adjb_s[pl.ds(start, tm), :] = ab
        deg = jnp.sum(a, axis=1, keepdims=True) + 1.0
        dk = jax.lax.rsqrt(deg)
        d_s[pl.ds(start, tm), :] = dk
        s1 = jnp.dot(x_ref[...].astype(jnp.bfloat16),
                     w1_ref[...].astype(jnp.bfloat16),
                     preferred_element_type=jnp.float32)
        t1k = (s1 * dk).astype(jnp.bfloat16)
        q = n // 4
        cs = [jax.lax.dot_general(
                  ab[:, j * q:(j + 1) * q], t1k, (((0,), (0,)), ((), ())),
                  preferred_element_type=jnp.float32)
              for j in range(4)]

        @pl.when(k == 0)
        def _init():
            for j in range(4):
                u_s[j * q:(j + 1) * q, :] = cs[j]

        @pl.when(k > 0)
        def _acc():
            for j in range(4):
                u_s[j * q:(j + 1) * q, :] += cs[j]

    @pl.when(p == 1)
    def _phase1():
        @pl.when(k == 0)
        def _compute_t2():
            d_all = d_s[...]
            h = jnp.maximum(d_all * u_s[...] + b1_ref[...], 0.0)
            s2 = jnp.dot(h.astype(jnp.bfloat16),
                         w2_ref[...].astype(jnp.bfloat16),
                         preferred_element_type=jnp.float32)
            t2_s[...] = (d_all * s2).astype(jnp.bfloat16)

        hm = tm // 2
        t2 = t2_s[...]
        a_lo = adjb_s[pl.ds(start, hm), :]
        acc_lo = jnp.dot(a_lo, t2, preferred_element_type=jnp.float32)
        o_ref[:hm, :] = d_s[pl.ds(start, hm), :] * acc_lo + b2_ref[...]
        a_hi = adjb_s[pl.ds(start + hm, hm), :]
        acc_hi = jnp.dot(a_hi, t2, preferred_element_type=jnp.float32)
        o_ref[hm:, :] = d_s[pl.ds(start + hm, hm), :] * acc_hi + b2_ref[...]


def kernel(adj, x, w1, b1, w2, b2):
    n = adj.shape[0]
    f_in, h_dim = w1.shape
    c_dim = w2.shape[1]
    fp = _round_up(f_in, 128)
    hp = _round_up(h_dim, 128)
    tm = _pick_tile(n, 512)
    nb = n // tm
    f32 = jnp.float32
    bf16 = jnp.bfloat16

    if f_in != fp or h_dim != hp:
        w1_in = jnp.zeros((fp, hp), f32).at[:f_in, :h_dim].set(w1)
    else:
        w1_in = w1
    x_in = x if f_in == fp else jnp.zeros((n, fp), f32).at[:, :f_in].set(x)
    if h_dim != hp:
        w2 = jnp.zeros((hp, c_dim), f32).at[:h_dim, :].set(w2)
        b1 = jnp.zeros((hp,), f32).at[:h_dim].set(b1.astype(f32))
    w2_in = w2
    b1_2d = b1.reshape(1, hp).astype(f32)
    b2_2d = b2.reshape(1, c_dim).astype(f32)

    mib = 1 << 20

    out = pl.pallas_call(
        functools.partial(_mega_kernel, tm=tm, nb=nb),
        out_shape=jax.ShapeDtypeStruct((n, c_dim), f32),
        grid_spec=pltpu.PrefetchScalarGridSpec(
            num_scalar_prefetch=0,
            grid=(2, nb),
            in_specs=[
                pl.BlockSpec((tm, n), lambda p, k: (jnp.where(p == 0, k, nb - 1), 0)),
                pl.BlockSpec((tm, fp), lambda p, k: (jnp.where(p == 0, k, nb - 1), 0)),
                pl.BlockSpec((fp, hp), lambda p, k: (0, 0)),
                pl.BlockSpec((1, hp), lambda p, k: (0, 0)),
                pl.BlockSpec((hp, c_dim), lambda p, k: (0, 0)),
                pl.BlockSpec((1, c_dim), lambda p, k: (0, 0)),
            ],
            out_specs=pl.BlockSpec((tm, c_dim), lambda p, k: (jnp.where(p == 1, k, 0), 0)),
            scratch_shapes=[
                pltpu.VMEM((n, n), bf16),
                pltpu.VMEM((n, 1), f32),
                pltpu.VMEM((n, hp), f32),
                pltpu.VMEM((n, c_dim), bf16),
            ],
        ),
        compiler_params=pltpu.CompilerParams(
            dimension_semantics=("arbitrary", "arbitrary"),
            vmem_limit_bytes=63 * mib,
        ),
    )(adj, x_in, w1_in, b1_2d, w2_in, b2_2d)

    return out
